# Initial kernel scaffold; baseline (speedup 1.0000x reference)
import functools

import jax
import jax.numpy as jnp
from jax import lax
from jax.experimental import pallas as pl
from jax.experimental.pallas import tpu as pltpu
from jax.experimental.pallas import tpu_sc as plsc

N_IDX = 81920
EMBED_DIM = 64
N_FEATURES = 4

_NUM_CORES = 2
_NUM_SUBCORES = 16
_NW = _NUM_CORES * _NUM_SUBCORES
_PER_W = N_IDX // _NW
_CHUNK = 1280
_N_CHUNKS = _PER_W // _CHUNK

_mesh = plsc.VectorSubcoreMesh(core_axis_name="c", subcore_axis_name="s")


@functools.partial(
    pl.kernel,
    mesh=_mesh,
    out_type=jax.ShapeDtypeStruct((N_FEATURES * N_IDX, EMBED_DIM), jnp.float32),
    scratch_types=[
        pltpu.VMEM((_CHUNK,), jnp.int32),
        pltpu.VMEM((_CHUNK, EMBED_DIM), jnp.float32),
        pltpu.SemaphoreType.DMA,
    ],
)
def _gather_kernel(idx0, idx1, idx2, idx3, t0, t1, t2, t3, out, idx_v, rows_v, sem):
    wid = lax.axis_index("s") * _NUM_CORES + lax.axis_index("c")
    base = wid * _PER_W
    for f, (idx_hbm, tab_hbm) in enumerate(
        ((idx0, t0), (idx1, t1), (idx2, t2), (idx3, t3))
    ):
        for c in range(_N_CHUNKS):
            off = base + c * _CHUNK
            pltpu.sync_copy(idx_hbm.at[pl.ds(off, _CHUNK)], idx_v)
            pltpu.async_copy(tab_hbm.at[idx_v], rows_v, sem).wait()
            pltpu.sync_copy(rows_v, out.at[pl.ds(f * N_IDX + off, _CHUNK)])


def kernel(idx0, idx1, idx2, idx3, table0, table1, table2, table3):
    return _gather_kernel(idx0, idx1, idx2, idx3, table0, table1, table2, table3)


# speedup vs baseline: 1.1270x; 1.1270x over previous
import functools

import jax
import jax.numpy as jnp
from jax import lax
from jax.experimental import pallas as pl
from jax.experimental.pallas import tpu as pltpu
from jax.experimental.pallas import tpu_sc as plsc

N_IDX = 81920
EMBED_DIM = 64
N_FEATURES = 4

_NUM_CORES = 2
_NUM_SUBCORES = 16
_NW = _NUM_CORES * _NUM_SUBCORES
_PER_W = N_IDX // _NW
_CHUNK = 1280
_N_CHUNKS = _PER_W // _CHUNK

_mesh = plsc.VectorSubcoreMesh(core_axis_name="c", subcore_axis_name="s")


@functools.partial(
    pl.kernel,
    mesh=_mesh,
    out_type=jax.ShapeDtypeStruct((N_FEATURES * N_IDX, EMBED_DIM), jnp.float32),
    scratch_types=[
        pltpu.VMEM((_CHUNK,), jnp.int32),
        pltpu.VMEM((_CHUNK, EMBED_DIM), jnp.float32),
        pltpu.SemaphoreType.DMA,
    ],
    compiler_params=pltpu.CompilerParams(use_tc_tiling_on_sc=False),
)
def _gather_kernel(idx0, idx1, idx2, idx3, t0, t1, t2, t3, out, idx_v, rows_v, sem):
    wid = lax.axis_index("s") * _NUM_CORES + lax.axis_index("c")
    base = wid * _PER_W
    for f, (idx_hbm, tab_hbm) in enumerate(
        ((idx0, t0), (idx1, t1), (idx2, t2), (idx3, t3))
    ):
        for c in range(_N_CHUNKS):
            off = base + c * _CHUNK
            pltpu.sync_copy(idx_hbm.at[pl.ds(off, _CHUNK)], idx_v)
            pltpu.async_copy(tab_hbm.at[idx_v], rows_v, sem).wait()
            pltpu.sync_copy(rows_v, out.at[pl.ds(f * N_IDX + off, _CHUNK)])


def kernel(idx0, idx1, idx2, idx3, table0, table1, table2, table3):
    return _gather_kernel(idx0, idx1, idx2, idx3, table0, table1, table2, table3)


# speedup vs baseline: 1.1318x; 1.0043x over previous
import functools

import jax
import jax.numpy as jnp
from jax import lax
from jax.experimental import pallas as pl
from jax.experimental.pallas import tpu as pltpu
from jax.experimental.pallas import tpu_sc as plsc

N_IDX = 81920
EMBED_DIM = 64
N_FEATURES = 4

_NUM_CORES = 2
_NUM_SUBCORES = 16
_NW = _NUM_CORES * _NUM_SUBCORES
_PER_W = N_IDX // _NW
_CHUNK = 512
_CPF = _PER_W // _CHUNK
_NCH = N_FEATURES * _CPF
_NBUF = 3

_mesh = plsc.VectorSubcoreMesh(core_axis_name="c", subcore_axis_name="s")


@functools.partial(
    pl.kernel,
    mesh=_mesh,
    out_type=jax.ShapeDtypeStruct((N_FEATURES * N_IDX, EMBED_DIM), jnp.float32),
    scratch_types=[
        pltpu.VMEM((N_FEATURES * _PER_W,), jnp.int32),
        [pltpu.VMEM((_CHUNK, EMBED_DIM), jnp.float32) for _ in range(_NBUF)],
        [pltpu.SemaphoreType.DMA for _ in range(_NBUF)],
        [pltpu.SemaphoreType.DMA for _ in range(_NBUF)],
    ],
    compiler_params=pltpu.CompilerParams(use_tc_tiling_on_sc=False),
)
def _gather_kernel(
    idx0, idx1, idx2, idx3, t0, t1, t2, t3, out, idx_all, rows, gsem, ssem
):
    wid = lax.axis_index("s") * _NUM_CORES + lax.axis_index("c")
    base = wid * _PER_W
    tabs = (t0, t1, t2, t3)

    for f, idx_hbm in enumerate((idx0, idx1, idx2, idx3)):
        pltpu.sync_copy(
            idx_hbm.at[pl.ds(base, _PER_W)], idx_all.at[pl.ds(f * _PER_W, _PER_W)]
        )

    def start_gather(c, b):
        f = c // _CPF
        return pltpu.async_copy(
            tabs[f].at[idx_all.at[pl.ds(c * _CHUNK, _CHUNK)]], rows[b], gsem[b]
        )

    def start_store(c, b):
        f, k = c // _CPF, c % _CPF
        dst = out.at[pl.ds(f * N_IDX + base + k * _CHUNK, _CHUNK)]
        return pltpu.async_copy(rows[b], dst, ssem[b])

    gathers = [None] * _NBUF
    stores = [None] * _NBUF
    for b in range(_NBUF):
        gathers[b] = start_gather(b, b)
    for c in range(_NCH):
        b = c % _NBUF
        gathers[b].wait()
        stores[b] = start_store(c, b)
        nxt = c + _NBUF
        if nxt < _NCH:
            stores[b].wait()
            gathers[b] = start_gather(nxt, b)
    for c in range(_NCH - _NBUF, _NCH):
        stores[c % _NBUF].wait()


def kernel(idx0, idx1, idx2, idx3, table0, table1, table2, table3):
    return _gather_kernel(idx0, idx1, idx2, idx3, table0, table1, table2, table3)


# speedup vs baseline: 1.3830x; 1.2219x over previous
import functools

import jax
import jax.numpy as jnp
from jax import lax
from jax.experimental import pallas as pl
from jax.experimental.pallas import tpu as pltpu
from jax.experimental.pallas import tpu_sc as plsc

N_IDX = 81920
EMBED_DIM = 64
N_FEATURES = 4

_NUM_CORES = 2
_NUM_SUBCORES = 16
_NW = _NUM_CORES * _NUM_SUBCORES
_PER_W = N_IDX // _NW
_CHUNK = 512
_CPF = _PER_W // _CHUNK
_GROUPS = _CHUNK // 16

_mesh = plsc.VectorSubcoreMesh(core_axis_name="c", subcore_axis_name="s")


@functools.partial(
    pl.kernel,
    mesh=_mesh,
    out_type=jax.ShapeDtypeStruct((N_FEATURES * N_IDX, EMBED_DIM), jnp.float32),
    scratch_types=[
        pltpu.VMEM((N_FEATURES * _PER_W,), jnp.int32),
        pltpu.VMEM((_CHUNK, EMBED_DIM), jnp.float32),
        pltpu.SemaphoreType.DMA,
    ],
)
def _gather_kernel(idx0, idx1, idx2, idx3, t0, t1, t2, t3, out, idx_all, rows, gsem):
    wid = lax.axis_index("s") * _NUM_CORES + lax.axis_index("c")
    base = wid * _PER_W

    for f, idx_hbm in enumerate((idx0, idx1, idx2, idx3)):
        pltpu.sync_copy(
            idx_hbm.at[pl.ds(base, _PER_W)], idx_all.at[pl.ds(f * _PER_W, _PER_W)]
        )

    for f, tab in enumerate((t0, t1, t2, t3)):
        def chunk_body(c, _, tab=tab, f=f):
            def group_body(g, _):
                idx_v = idx_all[pl.ds(f * _PER_W + c * _CHUNK + g * 16, 16)]
                for j in range(16):
                    r = jax.lax.squeeze(jax.lax.slice(idx_v, (j,), (j + 1,)), (0,))
                    pltpu.make_async_copy(
                        tab.at[pl.ds(r, 1)], rows.at[pl.ds(g * 16 + j, 1)], gsem
                    ).start()
                return 0

            lax.fori_loop(0, _GROUPS, group_body, 0)
            pltpu.make_async_copy(tab.at[pl.ds(0, _CHUNK)], rows, gsem).wait()
            pltpu.sync_copy(
                rows, out.at[pl.ds(f * N_IDX + base + c * _CHUNK, _CHUNK)]
            )
            return 0

        lax.fori_loop(0, _CPF, chunk_body, 0)


def kernel(idx0, idx1, idx2, idx3, table0, table1, table2, table3):
    return _gather_kernel(idx0, idx1, idx2, idx3, table0, table1, table2, table3)
